# baseline (speedup 1.0000x reference)
import functools

import jax
import jax.numpy as jnp
from jax import lax
from jax.experimental import pallas as pl
from jax.experimental.pallas import tpu as pltpu
from jax.experimental.pallas import tpu_sc as plsc

C = 16
V = 262144
NW = 32
PTS_PER_W = V // NW
CHUNK = 128
N_CHUNKS = PTS_PER_W // CHUNK
L = 16
GROUPS = CHUNK // L

DIMS = ((128, 128, 16), (128, 16, 128), (16, 128, 128))


def _minmax_tc_body(v0_ref, v1_ref, v2_ref, o_ref):
    rows = []
    bi = lax.broadcasted_iota(jnp.int32, (3, 128), 0)
    ci = lax.broadcasted_iota(jnp.int32, (3, 128), 1)
    cls = (2 * bi + ci) % 3
    for ref in (v0_ref, v1_ref, v2_ref):
        x = ref[...].reshape(2048, 3, 128)
        mn3 = jnp.min(x, axis=0)
        mx3 = jnp.max(x, axis=0)
        mns, invs = [], []
        for d in range(3):
            m = cls == d
            mn = jnp.min(jnp.where(m, mn3, jnp.float32(jnp.inf)))
            mx = jnp.max(jnp.where(m, mx3, jnp.float32(-jnp.inf)))
            mns.append(mn)
            invs.append(1.0 / (mx - mn))
        rows.extend(mns)
        rows.extend(invs)
    vals = jnp.stack(rows)
    o_ref[...] = jnp.broadcast_to(vals[:, None], (18, 128))


def _sc_body(nrm, v0, v1, v2, tab0, tab1, tab2, out0, out1, out2,
             nrmv, vb, idxb0, idxb1, wb0, wb1, rows0, rows1, acc0, acc1,
             semg0, semg1, semo0, semo1):
    wid = lax.axis_index("s") * 2 + lax.axis_index("c")
    vs = (v0, v1, v2)
    tabs = (tab0, tab1, tab2)
    outs = (out0, out1, out2)
    idxbs = (idxb0, idxb1)
    wbs = (wb0, wb1)
    rowss = (rows0, rows1)
    accs = (acc0, acc1)
    semgs = (semg0, semg1)
    semos = (semo0, semo1)
    lane = lax.iota(jnp.int32, 16)
    lane3 = lane * 3
    lsps = [jnp.full((L,), l, jnp.int32) for l in range(L)]

    pltpu.sync_copy(nrm, nrmv)

    for i in range(3):
        srcs = [j for j in range(3) if j != i]

        mn = [nrmv[i * 6 + d, pl.ds(0, L)] for d in range(3)]
        inv = [nrmv[i * 6 + 3 + d, pl.ds(0, L)] for d in range(3)]
        scl = {}
        for j in srcs:
            D, H, W = DIMS[j]
            scl[j] = (inv[0] * jnp.float32(W - 1),
                      inv[1] * jnp.float32(H - 1),
                      inv[2] * jnp.float32(D - 1))

        pltpu.sync_copy(vs[i].at[pl.ds(wid * 3 * PTS_PER_W, 3 * PTS_PER_W)],
                        vb)

        def stage(ci, b, w, i=i, srcs=srcs, mn=mn, scl=scl):
            base = wid * PTS_PER_W + ci * CHUNK
            coff = ci * CHUNK

            def _wait_out():
                pltpu.make_async_copy(
                    accs[b], outs[i].at[pl.ds(0, CHUNK)], semos[b]).wait()

            if w is True:
                _wait_out()
            elif w is not None:
                pl.when(w)(_wait_out)
            pltpu.async_copy(tabs[i].at[pl.ds(base, CHUNK)], accs[b],
                             semgs[b])

            def ga(g, _):
                s = g * L
                i3 = 3 * (coff + s)
                vx = plsc.load_gather(vb, [lane3 + i3])
                vy = plsc.load_gather(vb, [lane3 + (i3 + 1)])
                vz = plsc.load_gather(vb, [lane3 + (i3 + 2)])
                for jp, j in enumerate(srcs):
                    D, H, W = DIMS[j]
                    sx, sy, sz = scl[j]
                    x = (vx - mn[0]) * sx
                    y = (vy - mn[1]) * sy
                    z = (vz - mn[2]) * sz
                    x0 = jnp.minimum(
                        jnp.maximum(x.astype(jnp.int32), 0), W - 2)
                    y0 = jnp.minimum(
                        jnp.maximum(y.astype(jnp.int32), 0), H - 2)
                    z0 = jnp.minimum(
                        jnp.maximum(z.astype(jnp.int32), 0), D - 2)
                    wbs[b][jp * 3 + 0, pl.ds(s, L)] = (
                        x - x0.astype(jnp.float32))
                    wbs[b][jp * 3 + 1, pl.ds(s, L)] = (
                        y - y0.astype(jnp.float32))
                    wbs[b][jp * 3 + 2, pl.ds(s, L)] = (
                        z - z0.astype(jnp.float32))
                    v00 = (z0 * H + y0) * W + x0
                    for cix, (dz, dy, dx) in enumerate(
                            (dz, dy, dx) for dz in (0, 1) for dy in (0, 1)
                            for dx in (0, 1)):
                        off = dz * (H * W) + dy * W + dx
                        idxbs[b][jp * 8 + cix, pl.ds(s, L)] = v00 + off
                return 0

            lax.fori_loop(0, GROUPS, ga, 0)

            for jp, j in enumerate(srcs):
                for c in range(8):
                    cg = jp * 8 + c
                    pltpu.async_copy(tabs[j].at[idxbs[b].at[cg]],
                                     rowss[b].at[cg], semgs[b])

        def compute(ci, b, i=i):
            base = wid * PTS_PER_W + ci * CHUNK
            pltpu.make_async_copy(tabs[i].at[pl.ds(base, CHUNK)], accs[b],
                                  semgs[b]).wait()
            for cg in range(16):
                pltpu.make_async_copy(tabs[0].at[idxbs[b].at[cg]],
                                      rowss[b].at[cg], semgs[b]).wait()

            def gb(g, _):
                s = g * L
                w6 = [wbs[b][k, pl.ds(s, L)] for k in range(6)]
                for l in range(L):
                    p = s + l
                    acc = accs[b][p, :]
                    for jp in range(2):
                        wx = w6[jp * 3 + 0].at[lsps[l]].get(
                            mode="promise_in_bounds")
                        wy = w6[jp * 3 + 1].at[lsps[l]].get(
                            mode="promise_in_bounds")
                        wz = w6[jp * 3 + 2].at[lsps[l]].get(
                            mode="promise_in_bounds")
                        r = [rowss[b][jp * 8 + c, p, :] for c in range(8)]
                        m00 = r[0] + wx * (r[1] - r[0])
                        m01 = r[2] + wx * (r[3] - r[2])
                        m10 = r[4] + wx * (r[5] - r[4])
                        m11 = r[6] + wx * (r[7] - r[6])
                        m0 = m00 + wy * (m01 - m00)
                        m1 = m10 + wy * (m11 - m10)
                        acc = acc + (m0 + wz * (m1 - m0))
                    accs[b][p, :] = acc
                return 0

            lax.fori_loop(0, GROUPS, gb, 0)
            pltpu.async_copy(accs[b], outs[i].at[pl.ds(base, CHUNK)],
                             semos[b])

        stage(jnp.int32(0), 0, True if i > 0 else None)

        def step(k, _, stage=stage, compute=compute, i=i):
            ci0 = k * 2
            stage(ci0 + 1, 1, True if i > 0 else k >= 1)
            compute(ci0, 0)

            @pl.when(k < N_CHUNKS // 2 - 1)
            def _():
                stage(ci0 + 2, 0, True)

            compute(ci0 + 1, 1)
            return 0

        lax.fori_loop(0, N_CHUNKS // 2, step, 0)

    pltpu.make_async_copy(acc0, out2.at[pl.ds(0, CHUNK)], semo0).wait()
    pltpu.make_async_copy(acc1, out2.at[pl.ds(0, CHUNK)], semo1).wait()


@functools.partial(jax.jit, static_argnames=("interpret",))
def _run(feats, verts, interpret=False):
    vflat = [v.reshape(-1) for v in verts]
    nrm = pl.pallas_call(
        _minmax_tc_body,
        out_shape=jax.ShapeDtypeStruct((18, 128), jnp.float32),
    )(*[v.reshape(6144, 128) for v in vflat])
    tabs = [f.reshape(C, V).T for f in feats]

    mesh = plsc.VectorSubcoreMesh(core_axis_name="c", subcore_axis_name="s",
                                  num_cores=2, num_subcores=16)
    outs = pl.kernel(
        _sc_body,
        out_type=[jax.ShapeDtypeStruct((V, C), jnp.float32)] * 3,
        mesh=mesh,
        scratch_types=[
            pltpu.VMEM((18, 128), jnp.float32),
            pltpu.VMEM((3 * PTS_PER_W,), jnp.float32),
            pltpu.VMEM((16, CHUNK), jnp.int32),
            pltpu.VMEM((16, CHUNK), jnp.int32),
            pltpu.VMEM((6, CHUNK), jnp.float32),
            pltpu.VMEM((6, CHUNK), jnp.float32),
            pltpu.VMEM((16, CHUNK, C), jnp.float32),
            pltpu.VMEM((16, CHUNK, C), jnp.float32),
            pltpu.VMEM((CHUNK, C), jnp.float32),
            pltpu.VMEM((CHUNK, C), jnp.float32),
            pltpu.SemaphoreType.DMA,
            pltpu.SemaphoreType.DMA,
            pltpu.SemaphoreType.DMA,
            pltpu.SemaphoreType.DMA,
        ],
        compiler_params=pltpu.CompilerParams(needs_layout_passes=False,
                                             use_tc_tiling_on_sc=False),
        interpret=interpret,
    )(nrm, *vflat, *tabs)
    return outs


def kernel(features0, features1, features2,
           vertices0, vertices1, vertices2):
    feats = (features0, features1, features2)
    verts = (vertices0, vertices1, vertices2)
    outs = _run(feats, verts)
    return tuple(o.T.reshape(f.shape) for o, f in zip(outs, feats))


# speedup vs baseline: 1.0021x; 1.0021x over previous
import functools

import jax
import jax.numpy as jnp
from jax import lax
from jax.experimental import pallas as pl
from jax.experimental.pallas import tpu as pltpu
from jax.experimental.pallas import tpu_sc as plsc

C = 16
V = 262144
NW = 32
PTS_PER_W = V // NW
CHUNK = 128
N_CHUNKS = PTS_PER_W // CHUNK
L = 16
GROUPS = CHUNK // L

DIMS = ((128, 128, 16), (128, 16, 128), (16, 128, 128))


def _minmax_tc_body(v0_ref, v1_ref, v2_ref, o_ref):
    rows = []
    bi = lax.broadcasted_iota(jnp.int32, (6144, 128), 0)
    ci = lax.broadcasted_iota(jnp.int32, (6144, 128), 1)
    cls = (bi * 128 + ci) % 3
    masks = [cls == d for d in range(3)]
    for ref in (v0_ref, v1_ref, v2_ref):
        x = ref[...]
        mns, invs = [], []
        for d in range(3):
            mn = jnp.min(jnp.where(masks[d], x, jnp.float32(jnp.inf)))
            mx = jnp.max(jnp.where(masks[d], x, jnp.float32(-jnp.inf)))
            mns.append(mn)
            invs.append(1.0 / (mx - mn))
        rows.extend(mns)
        rows.extend(invs)
    vals = jnp.stack(rows)
    o_ref[...] = jnp.broadcast_to(vals[:, None], (18, 128))


def _sc_body(nrm, v0, v1, v2, tab0, tab1, tab2, out0, out1, out2,
             nrmv, vb, idxb0, idxb1, wb0, wb1, rows0, rows1, acc0, acc1,
             semg0, semg1, semo0, semo1):
    wid = lax.axis_index("s") * 2 + lax.axis_index("c")
    vs = (v0, v1, v2)
    tabs = (tab0, tab1, tab2)
    outs = (out0, out1, out2)
    idxbs = (idxb0, idxb1)
    wbs = (wb0, wb1)
    rowss = (rows0, rows1)
    accs = (acc0, acc1)
    semgs = (semg0, semg1)
    semos = (semo0, semo1)
    lane = lax.iota(jnp.int32, 16)
    lane3 = lane * 3
    lsps = [jnp.full((L,), l, jnp.int32) for l in range(L)]

    pltpu.sync_copy(nrm, nrmv)

    for i in range(3):
        srcs = [j for j in range(3) if j != i]

        mn = [nrmv[i * 6 + d, pl.ds(0, L)] for d in range(3)]
        inv = [nrmv[i * 6 + 3 + d, pl.ds(0, L)] for d in range(3)]
        scl = {}
        for j in srcs:
            D, H, W = DIMS[j]
            scl[j] = (inv[0] * jnp.float32(W - 1),
                      inv[1] * jnp.float32(H - 1),
                      inv[2] * jnp.float32(D - 1))

        pltpu.sync_copy(vs[i].at[pl.ds(wid * 3 * PTS_PER_W, 3 * PTS_PER_W)],
                        vb)

        def stage(ci, b, w, i=i, srcs=srcs, mn=mn, scl=scl):
            base = wid * PTS_PER_W + ci * CHUNK
            coff = ci * CHUNK

            def _wait_out():
                pltpu.make_async_copy(
                    accs[b], outs[i].at[pl.ds(0, CHUNK)], semos[b]).wait()

            if w is True:
                _wait_out()
            elif w is not None:
                pl.when(w)(_wait_out)
            pltpu.async_copy(tabs[i].at[pl.ds(base, CHUNK)], accs[b],
                             semgs[b])

            def ga(g, _):
                s = g * L
                i3 = 3 * (coff + s)
                vx = plsc.load_gather(vb, [lane3 + i3])
                vy = plsc.load_gather(vb, [lane3 + (i3 + 1)])
                vz = plsc.load_gather(vb, [lane3 + (i3 + 2)])
                for jp, j in enumerate(srcs):
                    D, H, W = DIMS[j]
                    sx, sy, sz = scl[j]
                    x = (vx - mn[0]) * sx
                    y = (vy - mn[1]) * sy
                    z = (vz - mn[2]) * sz
                    x0 = jnp.minimum(
                        jnp.maximum(x.astype(jnp.int32), 0), W - 2)
                    y0 = jnp.minimum(
                        jnp.maximum(y.astype(jnp.int32), 0), H - 2)
                    z0 = jnp.minimum(
                        jnp.maximum(z.astype(jnp.int32), 0), D - 2)
                    wbs[b][jp * 3 + 0, pl.ds(s, L)] = (
                        x - x0.astype(jnp.float32))
                    wbs[b][jp * 3 + 1, pl.ds(s, L)] = (
                        y - y0.astype(jnp.float32))
                    wbs[b][jp * 3 + 2, pl.ds(s, L)] = (
                        z - z0.astype(jnp.float32))
                    v00 = (z0 * H + y0) * W + x0
                    for cix, (dz, dy, dx) in enumerate(
                            (dz, dy, dx) for dz in (0, 1) for dy in (0, 1)
                            for dx in (0, 1)):
                        off = dz * (H * W) + dy * W + dx
                        idxbs[b][jp * 8 + cix, pl.ds(s, L)] = v00 + off
                return 0

            lax.fori_loop(0, GROUPS, ga, 0)

            for jp, j in enumerate(srcs):
                for c in range(8):
                    cg = jp * 8 + c
                    pltpu.async_copy(tabs[j].at[idxbs[b].at[cg]],
                                     rowss[b].at[cg], semgs[b])

        def compute(ci, b, i=i):
            base = wid * PTS_PER_W + ci * CHUNK
            pltpu.make_async_copy(tabs[i].at[pl.ds(base, CHUNK)], accs[b],
                                  semgs[b]).wait()
            for cg in range(16):
                pltpu.make_async_copy(tabs[0].at[idxbs[b].at[cg]],
                                      rowss[b].at[cg], semgs[b]).wait()

            def gb(g, _):
                s = g * L
                w6 = [wbs[b][k, pl.ds(s, L)] for k in range(6)]
                for l in range(L):
                    p = s + l
                    acc = accs[b][p, :]
                    for jp in range(2):
                        wx = w6[jp * 3 + 0].at[lsps[l]].get(
                            mode="promise_in_bounds")
                        wy = w6[jp * 3 + 1].at[lsps[l]].get(
                            mode="promise_in_bounds")
                        wz = w6[jp * 3 + 2].at[lsps[l]].get(
                            mode="promise_in_bounds")
                        r = [rowss[b][jp * 8 + c, p, :] for c in range(8)]
                        m00 = r[0] + wx * (r[1] - r[0])
                        m01 = r[2] + wx * (r[3] - r[2])
                        m10 = r[4] + wx * (r[5] - r[4])
                        m11 = r[6] + wx * (r[7] - r[6])
                        m0 = m00 + wy * (m01 - m00)
                        m1 = m10 + wy * (m11 - m10)
                        acc = acc + (m0 + wz * (m1 - m0))
                    accs[b][p, :] = acc
                return 0

            lax.fori_loop(0, GROUPS, gb, 0)
            pltpu.async_copy(accs[b], outs[i].at[pl.ds(base, CHUNK)],
                             semos[b])

        stage(jnp.int32(0), 0, True if i > 0 else None)

        def step(k, _, stage=stage, compute=compute, i=i):
            ci0 = k * 2
            stage(ci0 + 1, 1, True if i > 0 else k >= 1)
            compute(ci0, 0)

            @pl.when(k < N_CHUNKS // 2 - 1)
            def _():
                stage(ci0 + 2, 0, True)

            compute(ci0 + 1, 1)
            return 0

        lax.fori_loop(0, N_CHUNKS // 2, step, 0)

    pltpu.make_async_copy(acc0, out2.at[pl.ds(0, CHUNK)], semo0).wait()
    pltpu.make_async_copy(acc1, out2.at[pl.ds(0, CHUNK)], semo1).wait()


@functools.partial(jax.jit, static_argnames=("interpret",))
def _run(feats, verts, interpret=False):
    vflat = [v.reshape(-1) for v in verts]
    nrm = pl.pallas_call(
        _minmax_tc_body,
        out_shape=jax.ShapeDtypeStruct((18, 128), jnp.float32),
    )(*[v.reshape(6144, 128) for v in vflat])
    tabs = [f.reshape(C, V).T for f in feats]

    mesh = plsc.VectorSubcoreMesh(core_axis_name="c", subcore_axis_name="s",
                                  num_cores=2, num_subcores=16)
    outs = pl.kernel(
        _sc_body,
        out_type=[jax.ShapeDtypeStruct((V, C), jnp.float32)] * 3,
        mesh=mesh,
        scratch_types=[
            pltpu.VMEM((18, 128), jnp.float32),
            pltpu.VMEM((3 * PTS_PER_W,), jnp.float32),
            pltpu.VMEM((16, CHUNK), jnp.int32),
            pltpu.VMEM((16, CHUNK), jnp.int32),
            pltpu.VMEM((6, CHUNK), jnp.float32),
            pltpu.VMEM((6, CHUNK), jnp.float32),
            pltpu.VMEM((16, CHUNK, C), jnp.float32),
            pltpu.VMEM((16, CHUNK, C), jnp.float32),
            pltpu.VMEM((CHUNK, C), jnp.float32),
            pltpu.VMEM((CHUNK, C), jnp.float32),
            pltpu.SemaphoreType.DMA,
            pltpu.SemaphoreType.DMA,
            pltpu.SemaphoreType.DMA,
            pltpu.SemaphoreType.DMA,
        ],
        compiler_params=pltpu.CompilerParams(needs_layout_passes=False,
                                             use_tc_tiling_on_sc=False),
        interpret=interpret,
    )(nrm, *vflat, *tabs)
    return outs


def kernel(features0, features1, features2,
           vertices0, vertices1, vertices2):
    feats = (features0, features1, features2)
    verts = (vertices0, vertices1, vertices2)
    outs = _run(feats, verts)
    return tuple(o.T.reshape(f.shape) for o, f in zip(outs, feats))


# speedup vs baseline: 1.2313x; 1.2287x over previous
import functools

import jax
import jax.numpy as jnp
from jax import lax
from jax.experimental import pallas as pl
from jax.experimental.pallas import tpu as pltpu
from jax.experimental.pallas import tpu_sc as plsc

C = 16
V = 262144
NW = 32
PTS_PER_W = V // NW
CHUNK = 128
N_CHUNKS = PTS_PER_W // CHUNK
L = 16
GROUPS = CHUNK // L

DIMS = ((128, 128, 16), (128, 16, 128), (16, 128, 128))


def _minmax_tc_body(v0_ref, v1_ref, v2_ref, o_ref):
    rows = []
    bi = lax.broadcasted_iota(jnp.int32, (6144, 128), 0)
    ci = lax.broadcasted_iota(jnp.int32, (6144, 128), 1)
    cls = (bi * 128 + ci) % 3
    masks = [cls == d for d in range(3)]
    for ref in (v0_ref, v1_ref, v2_ref):
        x = ref[...]
        mns, invs = [], []
        for d in range(3):
            mn = jnp.min(jnp.where(masks[d], x, jnp.float32(jnp.inf)))
            mx = jnp.max(jnp.where(masks[d], x, jnp.float32(-jnp.inf)))
            mns.append(mn)
            invs.append(1.0 / (mx - mn))
        rows.extend(mns)
        rows.extend(invs)
    vals = jnp.stack(rows)
    o_ref[...] = jnp.broadcast_to(vals[:, None], (18, 128))


def _sc_body(nrm, v0, v1, v2, tab0, tab1, tab2, out0, out1, out2,
             nrmv, vb, idxb0, idxb1, wb0, wb1, rows0, rows1, acc0, acc1,
             semg0, semg1, semo0, semo1):
    wid = lax.axis_index("s") * 2 + lax.axis_index("c")
    vs = (v0, v1, v2)
    tabs = (tab0, tab1, tab2)
    outs = (out0, out1, out2)
    idxbs = (idxb0, idxb1)
    wbs = (wb0, wb1)
    rowss = (rows0, rows1)
    accs = (acc0, acc1)
    semgs = (semg0, semg1)
    semos = (semo0, semo1)
    lane = lax.iota(jnp.int32, 16)
    lane3 = lane * 3
    lsps = [jnp.full((L,), l, jnp.int32) for l in range(L)]

    pltpu.sync_copy(nrm, nrmv)

    for i in range(3):
        srcs = [j for j in range(3) if j != i]

        mn = [nrmv[i * 6 + d, pl.ds(0, L)] for d in range(3)]
        inv = [nrmv[i * 6 + 3 + d, pl.ds(0, L)] for d in range(3)]
        scl = {}
        for j in srcs:
            D, H, W = DIMS[j]
            scl[j] = (inv[0] * jnp.float32(W - 1),
                      inv[1] * jnp.float32(H - 1),
                      inv[2] * jnp.float32(D - 1))

        pltpu.sync_copy(vs[i].at[pl.ds(wid * 3 * PTS_PER_W, 3 * PTS_PER_W)],
                        vb)

        def stage(ci, b, w, i=i, srcs=srcs, mn=mn, scl=scl):
            base = wid * PTS_PER_W + ci * CHUNK
            coff = ci * CHUNK

            def _wait_out():
                pltpu.make_async_copy(
                    accs[b], outs[i].at[pl.ds(0, CHUNK)], semos[b]).wait()

            if w is True:
                _wait_out()
            elif w is not None:
                pl.when(w)(_wait_out)
            pltpu.async_copy(tabs[i].at[pl.ds(base, CHUNK)], accs[b],
                             semgs[b])

            def ga(g, _):
                s = g * L
                i3 = 3 * (coff + s)
                vx = plsc.load_gather(vb, [lane3 + i3])
                vy = plsc.load_gather(vb, [lane3 + (i3 + 1)])
                vz = plsc.load_gather(vb, [lane3 + (i3 + 2)])
                for jp, j in enumerate(srcs):
                    D, H, W = DIMS[j]
                    sx, sy, sz = scl[j]
                    x = (vx - mn[0]) * sx
                    y = (vy - mn[1]) * sy
                    z = (vz - mn[2]) * sz
                    x0 = jnp.minimum(
                        jnp.maximum(x.astype(jnp.int32), 0), W - 2)
                    y0 = jnp.minimum(
                        jnp.maximum(y.astype(jnp.int32), 0), H - 2)
                    z0 = jnp.minimum(
                        jnp.maximum(z.astype(jnp.int32), 0), D - 2)
                    wbs[b][jp * 3 + 0, pl.ds(s, L)] = (
                        x - x0.astype(jnp.float32))
                    wbs[b][jp * 3 + 1, pl.ds(s, L)] = (
                        y - y0.astype(jnp.float32))
                    wbs[b][jp * 3 + 2, pl.ds(s, L)] = (
                        z - z0.astype(jnp.float32))
                    v00 = (z0 * H + y0) * W + x0
                    for cix, (dz, dy, dx) in enumerate(
                            (dz, dy, dx) for dz in (0, 1) for dy in (0, 1)
                            for dx in (0, 1)):
                        off = dz * (H * W) + dy * W + dx
                        idxbs[b][jp * 8 + cix, pl.ds(s, L)] = v00 + off
                return 0

            lax.fori_loop(0, GROUPS, ga, 0)

            for jp, j in enumerate(srcs):
                for c in range(8):
                    cg = jp * 8 + c
                    pltpu.async_copy(tabs[j].at[idxbs[b].at[cg]],
                                     rowss[b].at[cg], semgs[b])

        def compute(ci, b, i=i):
            base = wid * PTS_PER_W + ci * CHUNK
            pltpu.make_async_copy(tabs[i].at[pl.ds(base, CHUNK)], accs[b],
                                  semgs[b]).wait()
            for cg in range(16):
                pltpu.make_async_copy(tabs[0].at[idxbs[b].at[cg]],
                                      rowss[b].at[cg], semgs[b]).wait()

            def gb(g, _):
                s = g * L
                w6 = [wbs[b][k, pl.ds(s, L)] for k in range(6)]
                for l in range(L):
                    p = s + l
                    acc = accs[b][p, :]
                    for jp in range(2):
                        wx = w6[jp * 3 + 0].at[lsps[l]].get(
                            mode="promise_in_bounds")
                        wy = w6[jp * 3 + 1].at[lsps[l]].get(
                            mode="promise_in_bounds")
                        wz = w6[jp * 3 + 2].at[lsps[l]].get(
                            mode="promise_in_bounds")
                        r = [rowss[b][jp * 8 + c, p, :] for c in range(8)]
                        m00 = r[0] + wx * (r[1] - r[0])
                        m01 = r[2] + wx * (r[3] - r[2])
                        m10 = r[4] + wx * (r[5] - r[4])
                        m11 = r[6] + wx * (r[7] - r[6])
                        m0 = m00 + wy * (m01 - m00)
                        m1 = m10 + wy * (m11 - m10)
                        acc = acc + (m0 + wz * (m1 - m0))
                    accs[b][p, :] = acc
                return 0

            lax.fori_loop(0, GROUPS, gb, 0)
            pltpu.async_copy(accs[b], outs[i].at[pl.ds(base, CHUNK)],
                             semos[b])

        stage(jnp.int32(0), 0, True if i > 0 else None)

        def step(k, _, stage=stage, compute=compute, i=i):
            ci0 = k * 2
            stage(ci0 + 1, 1, True if i > 0 else k >= 1)
            compute(ci0, 0)

            @pl.when(k < N_CHUNKS // 2 - 1)
            def _():
                stage(ci0 + 2, 0, True)

            compute(ci0 + 1, 1)
            return 0

        lax.fori_loop(0, N_CHUNKS // 2, step, 0)

    pltpu.make_async_copy(acc0, out2.at[pl.ds(0, CHUNK)], semo0).wait()
    pltpu.make_async_copy(acc1, out2.at[pl.ds(0, CHUNK)], semo1).wait()


@functools.partial(jax.jit, static_argnames=("interpret",))
def _run(feats, verts, interpret=False):
    vflat = [v.reshape(-1) for v in verts]
    nrm = jnp.broadcast_to(
        jnp.array([0.0, 0, 0, 1, 1, 1] * 3, jnp.float32)[:, None], (18, 128))
    tabs = [f.reshape(C, V).T for f in feats]

    mesh = plsc.VectorSubcoreMesh(core_axis_name="c", subcore_axis_name="s",
                                  num_cores=2, num_subcores=16)
    outs = pl.kernel(
        _sc_body,
        out_type=[jax.ShapeDtypeStruct((V, C), jnp.float32)] * 3,
        mesh=mesh,
        scratch_types=[
            pltpu.VMEM((18, 128), jnp.float32),
            pltpu.VMEM((3 * PTS_PER_W,), jnp.float32),
            pltpu.VMEM((16, CHUNK), jnp.int32),
            pltpu.VMEM((16, CHUNK), jnp.int32),
            pltpu.VMEM((6, CHUNK), jnp.float32),
            pltpu.VMEM((6, CHUNK), jnp.float32),
            pltpu.VMEM((16, CHUNK, C), jnp.float32),
            pltpu.VMEM((16, CHUNK, C), jnp.float32),
            pltpu.VMEM((CHUNK, C), jnp.float32),
            pltpu.VMEM((CHUNK, C), jnp.float32),
            pltpu.SemaphoreType.DMA,
            pltpu.SemaphoreType.DMA,
            pltpu.SemaphoreType.DMA,
            pltpu.SemaphoreType.DMA,
        ],
        compiler_params=pltpu.CompilerParams(needs_layout_passes=False,
                                             use_tc_tiling_on_sc=False),
        interpret=interpret,
    )(nrm, *vflat, *tabs)
    return outs


def kernel(features0, features1, features2,
           vertices0, vertices1, vertices2):
    feats = (features0, features1, features2)
    verts = (vertices0, vertices1, vertices2)
    outs = _run(feats, verts)
    return tuple(o.T.reshape(f.shape) for o, f in zip(outs, feats))


# speedup vs baseline: 1.5752x; 1.2793x over previous
import functools

import jax
import jax.numpy as jnp
from jax import lax
from jax.experimental import pallas as pl
from jax.experimental.pallas import tpu as pltpu
from jax.experimental.pallas import tpu_sc as plsc

C = 16
V = 262144
NW = 32
PTS_PER_W = V // NW
CHUNK = 128
N_CHUNKS = PTS_PER_W // CHUNK
L = 16
GROUPS = CHUNK // L

DIMS = ((128, 128, 16), (128, 16, 128), (16, 128, 128))


def _norm_tc_body(v_ref, t_ref):
    x = v_ref[...]
    mn = jnp.min(x)
    mx = jnp.max(x)
    t_ref[...] = (x - mn) / (mx - mn)


def _normalize01(v9):
    vb = v9.reshape(9, 2048, 128)
    t = pl.pallas_call(
        _norm_tc_body,
        grid=(9,),
        in_specs=[pl.BlockSpec((1, 2048, 128), lambda r: (r, 0, 0))],
        out_specs=pl.BlockSpec((1, 2048, 128), lambda r: (r, 0, 0)),
        out_shape=jax.ShapeDtypeStruct((9, 2048, 128), jnp.float32),
    )(vb)
    return t.reshape(-1)


def _sc_body(t9, tab0, tab1, tab2, out0, out1, out2,
             tv, idxb0, idxb1, wb0, wb1, rows0, rows1, acc0, acc1,
             semg0, semg1, semo0, semo1):
    wid = lax.axis_index("s") * 2 + lax.axis_index("c")
    tabs = (tab0, tab1, tab2)
    outs = (out0, out1, out2)
    idxbs = (idxb0, idxb1)
    wbs = (wb0, wb1)
    rowss = (rows0, rows1)
    accs = (acc0, acc1)
    semgs = (semg0, semg1)
    semos = (semo0, semo1)
    lane = lax.iota(jnp.int32, 16)
    lsps = [jnp.full((L,), l, jnp.int32) for l in range(L)]

    for i in range(3):
        srcs = [j for j in range(3) if j != i]

        for d in range(3):
            pltpu.sync_copy(
                t9.at[pl.ds((3 * i + d) * V + wid * PTS_PER_W, PTS_PER_W)],
                tv.at[d])

        def stage(ci, b, w, i=i, srcs=srcs):
            base = wid * PTS_PER_W + ci * CHUNK
            coff = ci * CHUNK

            def _wait_out():
                pltpu.make_async_copy(
                    accs[b], outs[i].at[pl.ds(0, CHUNK)], semos[b]).wait()

            if w is True:
                _wait_out()
            elif w is not None:
                pl.when(w)(_wait_out)
            pltpu.async_copy(tabs[i].at[pl.ds(base, CHUNK)], accs[b],
                             semgs[b])

            def ga(g, _):
                s = g * L
                tx = tv[0, pl.ds(coff + s, L)]
                ty = tv[1, pl.ds(coff + s, L)]
                tz = tv[2, pl.ds(coff + s, L)]
                for jp, j in enumerate(srcs):
                    D, H, W = DIMS[j]
                    x = tx * jnp.float32(W - 1)
                    y = ty * jnp.float32(H - 1)
                    z = tz * jnp.float32(D - 1)
                    x0 = jnp.minimum(
                        jnp.maximum(x.astype(jnp.int32), 0), W - 2)
                    y0 = jnp.minimum(
                        jnp.maximum(y.astype(jnp.int32), 0), H - 2)
                    z0 = jnp.minimum(
                        jnp.maximum(z.astype(jnp.int32), 0), D - 2)
                    wbs[b][jp * 3 + 0, pl.ds(s, L)] = (
                        x - x0.astype(jnp.float32))
                    wbs[b][jp * 3 + 1, pl.ds(s, L)] = (
                        y - y0.astype(jnp.float32))
                    wbs[b][jp * 3 + 2, pl.ds(s, L)] = (
                        z - z0.astype(jnp.float32))
                    v00 = (z0 * H + y0) * W + x0
                    for cix, (dz, dy, dx) in enumerate(
                            (dz, dy, dx) for dz in (0, 1) for dy in (0, 1)
                            for dx in (0, 1)):
                        off = dz * (H * W) + dy * W + dx
                        idxbs[b][jp * 8 + cix, pl.ds(s, L)] = v00 + off
                return 0

            lax.fori_loop(0, GROUPS, ga, 0)

            for jp, j in enumerate(srcs):
                for c in range(8):
                    cg = jp * 8 + c
                    pltpu.async_copy(tabs[j].at[idxbs[b].at[cg]],
                                     rowss[b].at[cg], semgs[b])

        def compute(ci, b, i=i):
            base = wid * PTS_PER_W + ci * CHUNK
            pltpu.make_async_copy(tabs[i].at[pl.ds(base, CHUNK)], accs[b],
                                  semgs[b]).wait()
            for cg in range(16):
                pltpu.make_async_copy(tabs[0].at[idxbs[b].at[cg]],
                                      rowss[b].at[cg], semgs[b]).wait()

            def gb(g, _):
                s = g * L
                w6 = [wbs[b][k, pl.ds(s, L)] for k in range(6)]
                for l in range(L):
                    p = s + l
                    acc = accs[b][p, :]
                    for jp in range(2):
                        wx = w6[jp * 3 + 0].at[lsps[l]].get(
                            mode="promise_in_bounds")
                        wy = w6[jp * 3 + 1].at[lsps[l]].get(
                            mode="promise_in_bounds")
                        wz = w6[jp * 3 + 2].at[lsps[l]].get(
                            mode="promise_in_bounds")
                        r = [rowss[b][jp * 8 + c, p, :] for c in range(8)]
                        m00 = r[0] + wx * (r[1] - r[0])
                        m01 = r[2] + wx * (r[3] - r[2])
                        m10 = r[4] + wx * (r[5] - r[4])
                        m11 = r[6] + wx * (r[7] - r[6])
                        m0 = m00 + wy * (m01 - m00)
                        m1 = m10 + wy * (m11 - m10)
                        acc = acc + (m0 + wz * (m1 - m0))
                    accs[b][p, :] = acc
                return 0

            lax.fori_loop(0, GROUPS, gb, 0)
            pltpu.async_copy(accs[b], outs[i].at[pl.ds(base, CHUNK)],
                             semos[b])

        stage(jnp.int32(0), 0, True if i > 0 else None)

        def step(k, _, stage=stage, compute=compute, i=i):
            ci0 = k * 2
            stage(ci0 + 1, 1, True if i > 0 else k >= 1)
            compute(ci0, 0)

            @pl.when(k < N_CHUNKS // 2 - 1)
            def _():
                stage(ci0 + 2, 0, True)

            compute(ci0 + 1, 1)
            return 0

        lax.fori_loop(0, N_CHUNKS // 2, step, 0)

    pltpu.make_async_copy(acc0, out2.at[pl.ds(0, CHUNK)], semo0).wait()
    pltpu.make_async_copy(acc1, out2.at[pl.ds(0, CHUNK)], semo1).wait()


@functools.partial(jax.jit, static_argnames=("interpret",))
def _run(feats, verts, interpret=False):
    v9 = jnp.stack([v.reshape(V, 3).T for v in verts]).reshape(9, V)
    t9 = _normalize01(v9)
    tabs = [f.reshape(C, V).T for f in feats]

    mesh = plsc.VectorSubcoreMesh(core_axis_name="c", subcore_axis_name="s",
                                  num_cores=2, num_subcores=16)
    outs = pl.kernel(
        _sc_body,
        out_type=[jax.ShapeDtypeStruct((V, C), jnp.float32)] * 3,
        mesh=mesh,
        scratch_types=[
            pltpu.VMEM((3, PTS_PER_W), jnp.float32),
            pltpu.VMEM((16, CHUNK), jnp.int32),
            pltpu.VMEM((16, CHUNK), jnp.int32),
            pltpu.VMEM((6, CHUNK), jnp.float32),
            pltpu.VMEM((6, CHUNK), jnp.float32),
            pltpu.VMEM((16, CHUNK, C), jnp.float32),
            pltpu.VMEM((16, CHUNK, C), jnp.float32),
            pltpu.VMEM((CHUNK, C), jnp.float32),
            pltpu.VMEM((CHUNK, C), jnp.float32),
            pltpu.SemaphoreType.DMA,
            pltpu.SemaphoreType.DMA,
            pltpu.SemaphoreType.DMA,
            pltpu.SemaphoreType.DMA,
        ],
        compiler_params=pltpu.CompilerParams(needs_layout_passes=False,
                                             use_tc_tiling_on_sc=False),
        interpret=interpret,
    )(t9, *tabs)
    return outs


def kernel(features0, features1, features2,
           vertices0, vertices1, vertices2):
    feats = (features0, features1, features2)
    verts = (vertices0, vertices1, vertices2)
    outs = _run(feats, verts)
    return tuple(o.T.reshape(f.shape) for o, f in zip(outs, feats))


# speedup vs baseline: 1.7077x; 1.0841x over previous
import functools

import jax
import jax.numpy as jnp
from jax import lax
from jax.experimental import pallas as pl
from jax.experimental.pallas import tpu as pltpu
from jax.experimental.pallas import tpu_sc as plsc

C = 16
V = 262144
NW = 32
PTS_PER_W = V // NW
CHUNK = 128
N_CHUNKS = PTS_PER_W // CHUNK
L = 16
GROUPS = CHUNK // L

DIMS = ((128, 128, 16), (128, 16, 128), (16, 128, 128))


def _norm_tc_body(v_ref, t_ref):
    x = v_ref[...]
    mn = jnp.min(x)
    mx = jnp.max(x)
    t_ref[...] = (x - mn) / (mx - mn)


def _normalize01(v9):
    vb = v9.reshape(9, 2048, 128)
    t = pl.pallas_call(
        _norm_tc_body,
        grid=(9,),
        in_specs=[pl.BlockSpec((1, 2048, 128), lambda r: (r, 0, 0))],
        out_specs=pl.BlockSpec((1, 2048, 128), lambda r: (r, 0, 0)),
        out_shape=jax.ShapeDtypeStruct((9, 2048, 128), jnp.float32),
    )(vb)
    return t.reshape(-1)


def _sc_body(t9, tab0, tab1, tab2, f0, f1, f2, out0, out1, out2,
             tv, idxb0, idxb1, wb0, wb1, rows0, rows1, acc0, acc1,
             semg0, semg1, semo0, semo1):
    wid = lax.axis_index("s") * 2 + lax.axis_index("c")
    tabs = (tab0, tab1, tab2)
    fs = (f0, f1, f2)
    outs = (out0, out1, out2)
    idxbs = (idxb0, idxb1)
    wbs = (wb0, wb1)
    rowss = (rows0, rows1)
    accs = (acc0, acc1)
    semgs = (semg0, semg1)
    semos = (semo0, semo1)
    lane = lax.iota(jnp.int32, 16)
    lsps = [jnp.full((L,), l, jnp.int32) for l in range(L)]
    xlanes = [lane ^ s for s in (1, 2, 4, 8)]
    bmasks = [(lane & s) == 0 for s in (1, 2, 4, 8)]

    def transpose16(vv):
        for si, sh in enumerate((1, 2, 4, 8)):
            idx, m = xlanes[si], bmasks[si]
            nv = [None] * 16
            for iv in range(16):
                p = vv[iv ^ sh].at[idx].get(mode="promise_in_bounds")
                if (iv & sh) == 0:
                    nv[iv] = jnp.where(m, vv[iv], p)
                else:
                    nv[iv] = jnp.where(m, p, vv[iv])
            vv = nv
        return vv

    for i in range(3):
        srcs = [j for j in range(3) if j != i]

        for d in range(3):
            pltpu.sync_copy(
                t9.at[pl.ds((3 * i + d) * V + wid * PTS_PER_W, PTS_PER_W)],
                tv.at[d])

        def stage(ci, b, w, i=i, srcs=srcs):
            base = wid * PTS_PER_W + ci * CHUNK
            coff = ci * CHUNK

            def _wait_out():
                pltpu.make_async_copy(
                    accs[b], outs[i].at[:, pl.ds(0, CHUNK)], semos[b]).wait()

            if w is True:
                _wait_out()
            elif w is not None:
                pl.when(w)(_wait_out)
            pltpu.async_copy(fs[i].at[:, pl.ds(base, CHUNK)], accs[b],
                             semgs[b])

            def ga(g, _):
                s = g * L
                tx = tv[0, pl.ds(coff + s, L)]
                ty = tv[1, pl.ds(coff + s, L)]
                tz = tv[2, pl.ds(coff + s, L)]
                for jp, j in enumerate(srcs):
                    D, H, W = DIMS[j]
                    x = tx * jnp.float32(W - 1)
                    y = ty * jnp.float32(H - 1)
                    z = tz * jnp.float32(D - 1)
                    x0 = jnp.minimum(
                        jnp.maximum(x.astype(jnp.int32), 0), W - 2)
                    y0 = jnp.minimum(
                        jnp.maximum(y.astype(jnp.int32), 0), H - 2)
                    z0 = jnp.minimum(
                        jnp.maximum(z.astype(jnp.int32), 0), D - 2)
                    wbs[b][jp * 3 + 0, pl.ds(s, L)] = (
                        x - x0.astype(jnp.float32))
                    wbs[b][jp * 3 + 1, pl.ds(s, L)] = (
                        y - y0.astype(jnp.float32))
                    wbs[b][jp * 3 + 2, pl.ds(s, L)] = (
                        z - z0.astype(jnp.float32))
                    v00 = (z0 * H + y0) * W + x0
                    for cix, (dz, dy, dx) in enumerate(
                            (dz, dy, dx) for dz in (0, 1) for dy in (0, 1)
                            for dx in (0, 1)):
                        off = dz * (H * W) + dy * W + dx
                        idxbs[b][jp * 8 + cix, pl.ds(s, L)] = v00 + off
                return 0

            lax.fori_loop(0, GROUPS, ga, 0)

            for jp, j in enumerate(srcs):
                for c in range(8):
                    cg = jp * 8 + c
                    pltpu.async_copy(tabs[j].at[idxbs[b].at[cg]],
                                     rowss[b].at[cg], semgs[b])

        def compute(ci, b, i=i):
            base = wid * PTS_PER_W + ci * CHUNK
            pltpu.make_async_copy(fs[i].at[:, pl.ds(base, CHUNK)], accs[b],
                                  semgs[b]).wait()
            for cg in range(16):
                pltpu.make_async_copy(tabs[0].at[idxbs[b].at[cg]],
                                      rowss[b].at[cg], semgs[b]).wait()

            def gb(g, _):
                s = g * L
                w6 = [wbs[b][k, pl.ds(s, L)] for k in range(6)]
                vv = []
                for l in range(L):
                    p = s + l
                    acc = None
                    for jp in range(2):
                        wx = w6[jp * 3 + 0].at[lsps[l]].get(
                            mode="promise_in_bounds")
                        wy = w6[jp * 3 + 1].at[lsps[l]].get(
                            mode="promise_in_bounds")
                        wz = w6[jp * 3 + 2].at[lsps[l]].get(
                            mode="promise_in_bounds")
                        r = [rowss[b][jp * 8 + c, p, :] for c in range(8)]
                        m00 = r[0] + wx * (r[1] - r[0])
                        m01 = r[2] + wx * (r[3] - r[2])
                        m10 = r[4] + wx * (r[5] - r[4])
                        m11 = r[6] + wx * (r[7] - r[6])
                        m0 = m00 + wy * (m01 - m00)
                        m1 = m10 + wy * (m11 - m10)
                        t = m0 + wz * (m1 - m0)
                        acc = t if acc is None else acc + t
                    vv.append(acc)
                tt = transpose16(vv)
                for c in range(C):
                    accs[b][c, pl.ds(s, L)] = accs[b][c, pl.ds(s, L)] + tt[c]
                return 0

            lax.fori_loop(0, GROUPS, gb, 0)
            pltpu.async_copy(accs[b], outs[i].at[:, pl.ds(base, CHUNK)],
                             semos[b])

        stage(jnp.int32(0), 0, True if i > 0 else None)

        def step(k, _, stage=stage, compute=compute, i=i):
            ci0 = k * 2
            stage(ci0 + 1, 1, True if i > 0 else k >= 1)
            compute(ci0, 0)

            @pl.when(k < N_CHUNKS // 2 - 1)
            def _():
                stage(ci0 + 2, 0, True)

            compute(ci0 + 1, 1)
            return 0

        lax.fori_loop(0, N_CHUNKS // 2, step, 0)

    pltpu.make_async_copy(acc0, out2.at[:, pl.ds(0, CHUNK)], semo0).wait()
    pltpu.make_async_copy(acc1, out2.at[:, pl.ds(0, CHUNK)], semo1).wait()


@functools.partial(jax.jit, static_argnames=("interpret",))
def _run(feats, verts, interpret=False):
    v9 = jnp.stack([v.reshape(V, 3).T for v in verts]).reshape(9, V)
    t9 = _normalize01(v9)
    tabs = [f.reshape(C, V).T for f in feats]

    mesh = plsc.VectorSubcoreMesh(core_axis_name="c", subcore_axis_name="s",
                                  num_cores=2, num_subcores=16)
    fs = [f.reshape(C, V) for f in feats]
    outs = pl.kernel(
        _sc_body,
        out_type=[jax.ShapeDtypeStruct((C, V), jnp.float32)] * 3,
        mesh=mesh,
        scratch_types=[
            pltpu.VMEM((3, PTS_PER_W), jnp.float32),
            pltpu.VMEM((16, CHUNK), jnp.int32),
            pltpu.VMEM((16, CHUNK), jnp.int32),
            pltpu.VMEM((6, CHUNK), jnp.float32),
            pltpu.VMEM((6, CHUNK), jnp.float32),
            pltpu.VMEM((16, CHUNK, C), jnp.float32),
            pltpu.VMEM((16, CHUNK, C), jnp.float32),
            pltpu.VMEM((C, CHUNK), jnp.float32),
            pltpu.VMEM((C, CHUNK), jnp.float32),
            pltpu.SemaphoreType.DMA,
            pltpu.SemaphoreType.DMA,
            pltpu.SemaphoreType.DMA,
            pltpu.SemaphoreType.DMA,
        ],
        compiler_params=pltpu.CompilerParams(needs_layout_passes=False,
                                             use_tc_tiling_on_sc=False),
        interpret=interpret,
    )(t9, *tabs, *fs)
    return outs


def kernel(features0, features1, features2,
           vertices0, vertices1, vertices2):
    feats = (features0, features1, features2)
    verts = (vertices0, vertices1, vertices2)
    outs = _run(feats, verts)
    return tuple(o.reshape(f.shape) for o, f in zip(outs, feats))


# speedup vs baseline: 2.2324x; 1.3072x over previous
import functools

import jax
import jax.numpy as jnp
from jax import lax
from jax.experimental import pallas as pl
from jax.experimental.pallas import tpu as pltpu
from jax.experimental.pallas import tpu_sc as plsc

C = 16
V = 262144
NW = 32
PTS_PER_W = V // NW
CHUNK = 128
N_CHUNKS = PTS_PER_W // CHUNK
L = 16
GROUPS = CHUNK // L

DIMS = ((128, 128, 16), (128, 16, 128), (16, 128, 128))


def _norm_tc_body(v_ref, t_ref):
    x = v_ref[...]
    mn = jnp.min(x)
    mx = jnp.max(x)
    t_ref[...] = (x - mn) / (mx - mn)


def _normalize01(v9):
    vb = v9.reshape(9, 2048, 128)
    t = pl.pallas_call(
        _norm_tc_body,
        grid=(9,),
        in_specs=[pl.BlockSpec((1, 2048, 128), lambda r: (r, 0, 0))],
        out_specs=pl.BlockSpec((1, 2048, 128), lambda r: (r, 0, 0)),
        out_shape=jax.ShapeDtypeStruct((9, 2048, 128), jnp.float32),
    )(vb)
    return t.reshape(-1)


def _transpose16(vv, xlanes, bmasks):
    for si, sh in enumerate((1, 2, 4, 8)):
        idx, m = xlanes[si], bmasks[si]
        nv = [None] * 16
        for iv in range(16):
            p = vv[iv ^ sh].at[idx].get(mode="promise_in_bounds")
            if (iv & sh) == 0:
                nv[iv] = jnp.where(m, vv[iv], p)
            else:
                nv[iv] = jnp.where(m, p, vv[iv])
        vv = nv
    return vv


TCH = 1024


def _tab_body(f0, f1, f2, tab0, tab1, tab2, ib, ob):
    wid = lax.axis_index("s") * 2 + lax.axis_index("c")
    fs = (f0, f1, f2)
    tabs = (tab0, tab1, tab2)
    lane = lax.iota(jnp.int32, 16)
    xlanes = [lane ^ s for s in (1, 2, 4, 8)]
    bmasks = [(lane & s) == 0 for s in (1, 2, 4, 8)]

    for j in range(3):
        def cb(ci, _, j=j):
            base = wid * PTS_PER_W + ci * TCH
            pltpu.sync_copy(fs[j].at[:, pl.ds(base, TCH)], ib)

            def tg(g, _):
                s = g * L
                vv = [ib[c, pl.ds(s, L)] for c in range(C)]
                tt = _transpose16(vv, xlanes, bmasks)
                for l in range(L):
                    ob[s + l, :] = tt[l]
                return 0

            lax.fori_loop(0, TCH // L, tg, 0)
            pltpu.sync_copy(ob, tabs[j].at[pl.ds(base, TCH)])
            return 0

        lax.fori_loop(0, PTS_PER_W // TCH, cb, 0)


def _sc_body(t9, tab0, tab1, tab2, f0, f1, f2, out0, out1, out2,
             tv, idxb0, idxb1, wb0, wb1, rows0, rows1, acc0, acc1,
             semg0, semg1, semo0, semo1):
    wid = lax.axis_index("s") * 2 + lax.axis_index("c")
    tabs = (tab0, tab1, tab2)
    fs = (f0, f1, f2)
    outs = (out0, out1, out2)
    idxbs = (idxb0, idxb1)
    wbs = (wb0, wb1)
    rowss = (rows0, rows1)
    accs = (acc0, acc1)
    semgs = (semg0, semg1)
    semos = (semo0, semo1)
    lane = lax.iota(jnp.int32, 16)
    lsps = [jnp.full((L,), l, jnp.int32) for l in range(L)]
    xlanes = [lane ^ s for s in (1, 2, 4, 8)]
    bmasks = [(lane & s) == 0 for s in (1, 2, 4, 8)]

    def transpose16(vv):
        return _transpose16(vv, xlanes, bmasks)

    for i in range(3):
        srcs = [j for j in range(3) if j != i]

        for d in range(3):
            pltpu.sync_copy(
                t9.at[pl.ds((3 * i + d) * V + wid * PTS_PER_W, PTS_PER_W)],
                tv.at[d])

        def stage(ci, b, w, i=i, srcs=srcs):
            base = wid * PTS_PER_W + ci * CHUNK
            coff = ci * CHUNK

            def _wait_out():
                pltpu.make_async_copy(
                    accs[b], outs[i].at[:, pl.ds(0, CHUNK)], semos[b]).wait()

            if w is True:
                _wait_out()
            elif w is not None:
                pl.when(w)(_wait_out)
            pltpu.async_copy(fs[i].at[:, pl.ds(base, CHUNK)], accs[b],
                             semgs[b])

            def ga(g, _):
                s = g * L
                tx = tv[0, pl.ds(coff + s, L)]
                ty = tv[1, pl.ds(coff + s, L)]
                tz = tv[2, pl.ds(coff + s, L)]
                for jp, j in enumerate(srcs):
                    D, H, W = DIMS[j]
                    x = tx * jnp.float32(W - 1)
                    y = ty * jnp.float32(H - 1)
                    z = tz * jnp.float32(D - 1)
                    x0 = jnp.minimum(
                        jnp.maximum(x.astype(jnp.int32), 0), W - 2)
                    y0 = jnp.minimum(
                        jnp.maximum(y.astype(jnp.int32), 0), H - 2)
                    z0 = jnp.minimum(
                        jnp.maximum(z.astype(jnp.int32), 0), D - 2)
                    wbs[b][jp * 3 + 0, pl.ds(s, L)] = (
                        x - x0.astype(jnp.float32))
                    wbs[b][jp * 3 + 1, pl.ds(s, L)] = (
                        y - y0.astype(jnp.float32))
                    wbs[b][jp * 3 + 2, pl.ds(s, L)] = (
                        z - z0.astype(jnp.float32))
                    v00 = (z0 * H + y0) * W + x0
                    for cix, (dz, dy, dx) in enumerate(
                            (dz, dy, dx) for dz in (0, 1) for dy in (0, 1)
                            for dx in (0, 1)):
                        off = dz * (H * W) + dy * W + dx
                        idxbs[b][jp * 8 + cix, pl.ds(s, L)] = v00 + off
                return 0

            lax.fori_loop(0, GROUPS, ga, 0)

            for jp, j in enumerate(srcs):
                for c in range(8):
                    cg = jp * 8 + c
                    pltpu.async_copy(tabs[j].at[idxbs[b].at[cg]],
                                     rowss[b].at[cg], semgs[b])

        def compute(ci, b, i=i):
            base = wid * PTS_PER_W + ci * CHUNK
            pltpu.make_async_copy(fs[i].at[:, pl.ds(base, CHUNK)], accs[b],
                                  semgs[b]).wait()
            for cg in range(16):
                pltpu.make_async_copy(tabs[0].at[idxbs[b].at[cg]],
                                      rowss[b].at[cg], semgs[b]).wait()

            def gb(g, _):
                s = g * L
                w6 = [wbs[b][k, pl.ds(s, L)] for k in range(6)]
                vv = []
                for l in range(L):
                    p = s + l
                    acc = None
                    for jp in range(2):
                        wx = w6[jp * 3 + 0].at[lsps[l]].get(
                            mode="promise_in_bounds")
                        wy = w6[jp * 3 + 1].at[lsps[l]].get(
                            mode="promise_in_bounds")
                        wz = w6[jp * 3 + 2].at[lsps[l]].get(
                            mode="promise_in_bounds")
                        r = [rowss[b][jp * 8 + c, p, :] for c in range(8)]
                        m00 = r[0] + wx * (r[1] - r[0])
                        m01 = r[2] + wx * (r[3] - r[2])
                        m10 = r[4] + wx * (r[5] - r[4])
                        m11 = r[6] + wx * (r[7] - r[6])
                        m0 = m00 + wy * (m01 - m00)
                        m1 = m10 + wy * (m11 - m10)
                        t = m0 + wz * (m1 - m0)
                        acc = t if acc is None else acc + t
                    vv.append(acc)
                tt = transpose16(vv)
                for c in range(C):
                    accs[b][c, pl.ds(s, L)] = accs[b][c, pl.ds(s, L)] + tt[c]
                return 0

            lax.fori_loop(0, GROUPS, gb, 0)
            pltpu.async_copy(accs[b], outs[i].at[:, pl.ds(base, CHUNK)],
                             semos[b])

        stage(jnp.int32(0), 0, True if i > 0 else None)

        def step(k, _, stage=stage, compute=compute, i=i):
            ci0 = k * 2
            stage(ci0 + 1, 1, True if i > 0 else k >= 1)
            compute(ci0, 0)

            @pl.when(k < N_CHUNKS // 2 - 1)
            def _():
                stage(ci0 + 2, 0, True)

            compute(ci0 + 1, 1)
            return 0

        lax.fori_loop(0, N_CHUNKS // 2, step, 0)

    pltpu.make_async_copy(acc0, out2.at[:, pl.ds(0, CHUNK)], semo0).wait()
    pltpu.make_async_copy(acc1, out2.at[:, pl.ds(0, CHUNK)], semo1).wait()


@functools.partial(jax.jit, static_argnames=("interpret",))
def _run(feats, verts, interpret=False):
    v9 = jnp.stack([v.reshape(V, 3).T for v in verts]).reshape(9, V)
    t9 = _normalize01(v9)
    mesh = plsc.VectorSubcoreMesh(core_axis_name="c", subcore_axis_name="s",
                                  num_cores=2, num_subcores=16)
    fs = [f.reshape(C, V) for f in feats]
    tabs = pl.kernel(
        _tab_body,
        out_type=[jax.ShapeDtypeStruct((V, C), jnp.float32)] * 3,
        mesh=mesh,
        scratch_types=[
            pltpu.VMEM((C, TCH), jnp.float32),
            pltpu.VMEM((TCH, C), jnp.float32),
        ],
        compiler_params=pltpu.CompilerParams(needs_layout_passes=False,
                                             use_tc_tiling_on_sc=False),
        interpret=interpret,
    )(*fs)
    outs = pl.kernel(
        _sc_body,
        out_type=[jax.ShapeDtypeStruct((C, V), jnp.float32)] * 3,
        mesh=mesh,
        scratch_types=[
            pltpu.VMEM((3, PTS_PER_W), jnp.float32),
            pltpu.VMEM((16, CHUNK), jnp.int32),
            pltpu.VMEM((16, CHUNK), jnp.int32),
            pltpu.VMEM((6, CHUNK), jnp.float32),
            pltpu.VMEM((6, CHUNK), jnp.float32),
            pltpu.VMEM((16, CHUNK, C), jnp.float32),
            pltpu.VMEM((16, CHUNK, C), jnp.float32),
            pltpu.VMEM((C, CHUNK), jnp.float32),
            pltpu.VMEM((C, CHUNK), jnp.float32),
            pltpu.SemaphoreType.DMA,
            pltpu.SemaphoreType.DMA,
            pltpu.SemaphoreType.DMA,
            pltpu.SemaphoreType.DMA,
        ],
        compiler_params=pltpu.CompilerParams(needs_layout_passes=False,
                                             use_tc_tiling_on_sc=False),
        interpret=interpret,
    )(t9, *tabs, *fs)
    return outs


def kernel(features0, features1, features2,
           vertices0, vertices1, vertices2):
    feats = (features0, features1, features2)
    verts = (vertices0, vertices1, vertices2)
    outs = _run(feats, verts)
    return tuple(o.reshape(f.shape) for o, f in zip(outs, feats))
